# baseline (speedup 1.0000x reference)
import functools

import jax
import jax.numpy as jnp
from jax import lax
from jax.experimental import pallas as pl
from jax.experimental.pallas import tpu as pltpu
from jax.experimental.pallas import tpu_sc as plsc

N = 10000
E = 160000
NC = 2
NS = 16
L = 16

N_PAD = 10240
E_PAD = NC * NS * 5120
EPT = E_PAD // (NC * NS)
CH = 128
NCHUNK = EPT // CH
ROWS_PER_TILE = N_PAD // NS
ZROWS = 64


def _mesh():
  return plsc.VectorSubcoreMesh(
      core_axis_name="c", subcore_axis_name="s", num_cores=NC,
      num_subcores=NS)


_SC_PARAMS = pltpu.CompilerParams(
    needs_layout_passes=False, use_tc_tiling_on_sc=False)


def _tile_edge_base(c, s):
  return (s * NC + c) * EPT


def _deg_body(src_hbm, wm16_hbm, out_hbm, src_v, val_v, zbuf, part):
  c = lax.axis_index("c")
  s = lax.axis_index("s")

  def zero_z(r, _):
    for q in range(1):
      zbuf[r, pl.ds(0, 16)] = jnp.zeros((16,), jnp.float32)
    return 0
  lax.fori_loop(0, ZROWS, zero_z, 0)
  for z in range(ROWS_PER_TILE // ZROWS):
    pltpu.sync_copy(zbuf, part.at[pl.ds(s * ROWS_PER_TILE + z * ZROWS, ZROWS)])
  plsc.subcore_barrier()

  ebase = _tile_edge_base(c, s)

  def chunk(j, _):
    off = ebase + j * CH
    pltpu.sync_copy(src_hbm.at[pl.ds(off, CH)], src_v)
    pltpu.sync_copy(wm16_hbm.at[pl.ds(off, CH)], val_v)
    pltpu.sync_copy(val_v, part.at[src_v], add=True)
    return 0
  lax.fori_loop(0, NCHUNK, chunk, 0)
  plsc.subcore_barrier()

  for z in range(ROWS_PER_TILE // ZROWS):
    r0 = s * ROWS_PER_TILE + z * ZROWS
    pltpu.sync_copy(part.at[pl.ds(r0, ZROWS)], out_hbm.at[c].at[pl.ds(r0, ZROWS)])


_deg_call = pl.kernel(
    _deg_body,
    out_type=jax.ShapeDtypeStruct((NC, N_PAD, 16), jnp.float32),
    mesh=_mesh(),
    compiler_params=_SC_PARAMS,
    scratch_types=[
        pltpu.VMEM((CH,), jnp.int32),
        pltpu.VMEM((CH, 16), jnp.float32),
        pltpu.VMEM((ZROWS, 16), jnp.float32),
        pltpu.VMEM_SHARED((N_PAD, 16), jnp.float32),
    ],
)


def _norm_body(src_hbm, dst_hbm, wm_hbm, dinv_hbm, nrm1_hbm, nrm2_hbm,
               src_v, dst_v, wm_v, n1_v, n2_v, dinv_v):
  c = lax.axis_index("c")
  s = lax.axis_index("s")
  pltpu.sync_copy(dinv_hbm, dinv_v)
  ebase = _tile_edge_base(c, s)

  def chunk(j, _):
    off = ebase + j * CH
    pltpu.sync_copy(src_hbm.at[pl.ds(off, CH)], src_v)
    pltpu.sync_copy(dst_hbm.at[pl.ds(off, CH)], dst_v)
    pltpu.sync_copy(wm_hbm.at[pl.ds(off, CH)], wm_v)

    def grp(g, _):
      s16 = src_v[pl.ds(g * 16, 16)]
      d16 = dst_v[pl.ds(g * 16, 16)]
      w16 = wm_v[pl.ds(g * 16, 16)]
      ds_ = plsc.load_gather(dinv_v, [s16])
      dd_ = plsc.load_gather(dinv_v, [d16])
      n1 = -(ds_ * w16 * dd_)
      n1_v[pl.ds(g * 16, 16)] = n1
      n2_v[pl.ds(g * 16, 16)] = n1 + n1
      return 0
    lax.fori_loop(0, CH // 16, grp, 0)
    pltpu.sync_copy(n1_v, nrm1_hbm.at[pl.ds(off, CH)])
    pltpu.sync_copy(n2_v, nrm2_hbm.at[pl.ds(off, CH)])
    return 0
  lax.fori_loop(0, NCHUNK, chunk, 0)


_norm_call = pl.kernel(
    _norm_body,
    out_type=(jax.ShapeDtypeStruct((E_PAD,), jnp.float32),
              jax.ShapeDtypeStruct((E_PAD,), jnp.float32)),
    mesh=_mesh(),
    compiler_params=_SC_PARAMS,
    scratch_types=[
        pltpu.VMEM((CH,), jnp.int32),
        pltpu.VMEM((CH,), jnp.int32),
        pltpu.VMEM((CH,), jnp.float32),
        pltpu.VMEM((CH,), jnp.float32),
        pltpu.VMEM((CH,), jnp.float32),
        pltpu.VMEM((N_PAD,), jnp.float32),
    ],
)


def _make_prop(F):
  def body(h_hbm, src_hbm, dst_hbm, nrm_hbm, out_hbm,
           sb0, sb1, sb2, db0, db1, db2, nx0, nx1, nx2, G, zbuf, part,
           ig0, ig1, ig2, gs0, gs1, ss0, ss1):
    c = lax.axis_index("c")
    s = lax.axis_index("s")
    srcb = (sb0, sb1, sb2)
    dstb = (db0, db1, db2)
    nxb = (nx0, nx1, nx2)
    isems = (ig0, ig1, ig2)
    gsems = (gs0, gs1)
    ssems = (ss0, ss1)

    def zero_z(r, _):
      for q in range(F // 16):
        zbuf[r, pl.ds(q * 16, 16)] = jnp.zeros((16,), jnp.float32)
      return 0
    lax.fori_loop(0, ZROWS, zero_z, 0)
    for z in range(ROWS_PER_TILE // ZROWS):
      pltpu.sync_copy(zbuf,
                      part.at[pl.ds(s * ROWS_PER_TILE + z * ZROWS, ZROWS)])
    plsc.subcore_barrier()

    ebase = _tile_edge_base(c, s)

    def start_idx(j, t):
      off = ebase + j * CH
      pltpu.async_copy(src_hbm.at[pl.ds(off, CH)], srcb[t], isems[t])
      pltpu.async_copy(dst_hbm.at[pl.ds(off, CH)], dstb[t], isems[t])
      pltpu.async_copy(nrm_hbm.at[pl.ds(off, CH), pl.ds(0, 16)],
                       nxb[t], isems[t])

    def wait_idx(t):
      pltpu.make_async_copy(src_hbm.at[pl.ds(0, CH)], srcb[t],
                            isems[t]).wait()
      pltpu.make_async_copy(dst_hbm.at[pl.ds(0, CH)], dstb[t],
                            isems[t]).wait()
      pltpu.make_async_copy(nrm_hbm.at[pl.ds(0, CH), pl.ds(0, 16)],
                            nxb[t], isems[t]).wait()

    def start_gather(t, b):
      pltpu.async_copy(h_hbm.at[srcb[t]], G.at[b], gsems[b])

    def wait_gather(t, b):
      pltpu.make_async_copy(h_hbm.at[srcb[0]], G.at[b], gsems[b]).wait()

    start_idx(0, 0)
    start_idx(1, 1)
    wait_idx(0)
    start_gather(0, 0)

    def chunk(j, _):
      b = lax.rem(j, 2)
      t = lax.rem(j, 3)
      t1 = lax.rem(j + 1, 3)
      t2 = lax.rem(j + 2, 3)

      for bb in range(2):
        @pl.when(b == bb)
        def _():
          wait_gather(t, bb)

          @pl.when(j >= 1)
          def _():
            pltpu.make_async_copy(G.at[1 - bb], part.at[dstb[0]],
                                  ssems[1 - bb]).wait()

          @pl.when(j + 2 <= NCHUNK - 1)
          def _():
            for tt in range(3):
              @pl.when(t2 == tt)
              def _():
                start_idx(j + 2, tt)

          @pl.when(j + 1 <= NCHUNK - 1)
          def _():
            for tt in range(3):
              @pl.when(t1 == tt)
              def _():
                wait_idx(tt)
                start_gather(tt, 1 - bb)

          for tt2 in range(3):
            @pl.when(t == tt2)
            def _():
              @plsc.parallel_loop(0, CH, 1, unroll=4)
              def _(e):
                row = nxb[tt2][e, pl.ds(0, 16)]
                for q in range(F // 16):
                  G[bb, e, pl.ds(q * 16, 16)] = (
                      G[bb, e, pl.ds(q * 16, 16)] * row)

          for tt in range(3):
            @pl.when(t == tt)
            def _():
              pltpu.async_copy(G.at[bb], part.at[dstb[tt]], ssems[bb],
                               add=True)
      return 0
    lax.fori_loop(0, NCHUNK, chunk, 0)

    lastb = (NCHUNK - 1) % 2
    pltpu.make_async_copy(G.at[lastb], part.at[dstb[0]],
                          ssems[lastb]).wait()
    plsc.subcore_barrier()

    for z in range(ROWS_PER_TILE // ZROWS):
      r0 = s * ROWS_PER_TILE + z * ZROWS
      pltpu.sync_copy(part.at[pl.ds(r0, ZROWS)],
                      out_hbm.at[c].at[pl.ds(r0, ZROWS)])

  return pl.kernel(
      body,
      out_type=jax.ShapeDtypeStruct((NC, N_PAD, F), jnp.float32),
      mesh=_mesh(),
      compiler_params=_SC_PARAMS,
      scratch_types=[
          pltpu.VMEM((CH,), jnp.int32),
          pltpu.VMEM((CH,), jnp.int32),
          pltpu.VMEM((CH,), jnp.int32),
          pltpu.VMEM((CH,), jnp.int32),
          pltpu.VMEM((CH,), jnp.int32),
          pltpu.VMEM((CH,), jnp.int32),
          pltpu.VMEM((CH, 16), jnp.float32),
          pltpu.VMEM((CH, 16), jnp.float32),
          pltpu.VMEM((CH, 16), jnp.float32),
          pltpu.VMEM((2, CH, F), jnp.float32),
          pltpu.VMEM((ZROWS, F), jnp.float32),
          pltpu.VMEM_SHARED((N_PAD, F), jnp.float32),
          pltpu.SemaphoreType.DMA,
          pltpu.SemaphoreType.DMA,
          pltpu.SemaphoreType.DMA,
          pltpu.SemaphoreType.DMA,
          pltpu.SemaphoreType.DMA,
          pltpu.SemaphoreType.DMA,
          pltpu.SemaphoreType.DMA,
      ],
  )


_prop128 = _make_prop(128)
_prop64 = _make_prop(64)
_prop_by_f = {128: _prop128, 64: _prop64}


def _dinv_body(dp_ref, o_ref):
  d = dp_ref[0, :, 0] + dp_ref[1, :, 0]
  o_ref[...] = jnp.where(
      d > 0.0, lax.rsqrt(jnp.maximum(d, 1e-12)), 0.0)


def _dinv_call(deg_parts):
  return pl.pallas_call(
      _dinv_body,
      out_shape=jax.ShapeDtypeStruct((N_PAD,), jnp.float32),
  )(deg_parts)


def _combine_body(coef_ref, p_ref, t0_ref, w_ref, acc_ref, t2_ref, acc2_ref):
  a = coef_ref[0, 0]
  b = coef_ref[0, 1]
  t2 = a * (p_ref[0] + p_ref[1]) - b * t0_ref[...]
  t2_ref[...] = t2
  acc2_ref[...] = acc_ref[...] + jnp.dot(
      t2, w_ref[...], preferred_element_type=jnp.float32)


def _combine(coef, parts, t0, w, acc):
  F = t0.shape[1]
  FO = w.shape[1]
  R = 1024
  grid = (N_PAD // R,)
  return pl.pallas_call(
      _combine_body,
      grid=grid,
      in_specs=[
          pl.BlockSpec(memory_space=pltpu.SMEM),
          pl.BlockSpec((NC, R, F), lambda i: (0, i, 0)),
          pl.BlockSpec((R, F), lambda i: (i, 0)),
          pl.BlockSpec((F, FO), lambda i: (0, 0)),
          pl.BlockSpec((R, FO), lambda i: (i, 0)),
      ],
      out_specs=[
          pl.BlockSpec((R, F), lambda i: (i, 0)),
          pl.BlockSpec((R, FO), lambda i: (i, 0)),
      ],
      out_shape=[
          jax.ShapeDtypeStruct((N_PAD, F), jnp.float32),
          jax.ShapeDtypeStruct((N_PAD, FO), jnp.float32),
      ],
  )(coef, parts, t0, w, acc)


def _silu_body(acc_ref, b_ref, o_ref):
  x = acc_ref[...] + b_ref[...]
  o_ref[...] = x * (1.0 / (1.0 + jnp.exp(-x)))


def _silu(acc, b2d):
  return pl.pallas_call(
      _silu_body,
      out_shape=jax.ShapeDtypeStruct(acc.shape, jnp.float32),
  )(acc, b2d)


def _sigmoid_body(acc_ref, o_ref):
  x = acc_ref[...]
  o_ref[...] = 1.0 / (1.0 + jnp.exp(-x))


def _sigmoid(acc):
  return pl.pallas_call(
      _sigmoid_body,
      out_shape=jax.ShapeDtypeStruct(acc.shape, jnp.float32),
  )(acc)


def _cheb_layer(h, w_stack, bias, src_p, dst_p, nrm1x, nrm2x):
  K, F, FO = w_stack.shape
  prop = _prop_by_f.get(F)
  zparts = jnp.zeros((NC, N_PAD, F), jnp.float32)
  zacc = jnp.zeros((N_PAD, FO), jnp.float32)
  c_id = jnp.array([[0.0, -1.0]], jnp.float32)
  c_t1 = jnp.array([[1.0, 0.0]], jnp.float32)
  c_rec = jnp.array([[2.0, 1.0]], jnp.float32)

  _, acc = _combine(c_id, zparts, h, w_stack[0], zacc)
  if K > 1:
    parts = prop(h, src_p, dst_p, nrm1x)
    t1, acc = _combine(c_t1, parts, h, w_stack[1], acc)
    if K > 2:
      def body(carry, wk):
        t0, t1, acc = carry
        parts = prop(t1, src_p, dst_p, nrm2x)
        t2, acc = _combine(c_rec, parts, t0, wk, acc)
        return (t1, t2, acc), None
      (_, _, acc), _ = lax.scan(body, (h, t1, acc), w_stack[2:])
  if bias is not None:
    return _silu(acc, bias)
  return acc


def _pad_w(w, fi, fo):
  K = w.shape[0]
  out = jnp.zeros((K, fi, fo), jnp.float32)
  return out.at[:, : w.shape[1], : w.shape[2]].set(w)


def kernel(x, edge_index, weight, W1, b1, W2, b2, W3, b3, W4):
  src = edge_index[0]
  dst = edge_index[1]
  wm = jnp.where(src != dst, weight, 0.0)

  pad = E_PAD - E
  src_p = jnp.concatenate([src, jnp.zeros((pad,), jnp.int32)])
  dst_p = jnp.concatenate([dst, jnp.zeros((pad,), jnp.int32)])
  wm_p = jnp.concatenate([wm, jnp.zeros((pad,), jnp.float32)])
  wm16 = jnp.broadcast_to(wm_p[:, None], (E_PAD, 16))

  deg_parts = _deg_call(src_p, wm16)
  dinv = _dinv_call(deg_parts)
  nrm1, nrm2 = _norm_call(src_p, dst_p, wm_p, dinv)
  nrm1x = jnp.broadcast_to(nrm1[:, None], (E_PAD, 16))
  nrm2x = jnp.broadcast_to(nrm2[:, None], (E_PAD, 16))

  h = jnp.zeros((N_PAD, 128), jnp.float32).at[:N].set(x)

  w1 = _pad_w(W1, 128, 64)
  w2 = _pad_w(W2, 64, 64)
  w3 = _pad_w(W3, 64, 32)
  w4 = _pad_w(W4, 32, 128)
  b1p = jnp.zeros((1, 64), jnp.float32).at[0, :60].set(b1)
  b2p = jnp.zeros((1, 64), jnp.float32).at[0, :60].set(b2)
  b3p = jnp.zeros((1, 32), jnp.float32).at[0, :30].set(b3)

  h = _cheb_layer(h, w1, b1p, src_p, dst_p, nrm1x, nrm2x)
  h = _cheb_layer(h, w2, b2p, src_p, dst_p, nrm1x, nrm2x)
  h = _cheb_layer(h, w3, b3p, src_p, dst_p, nrm1x, nrm2x)
  h = _cheb_layer(h, w4, None, src_p, dst_p, nrm1x, nrm2x)
  out = _sigmoid(h)
  return out[:N, 0:1]


# speedup vs baseline: 1.5624x; 1.5624x over previous
import functools

import jax
import jax.numpy as jnp
from jax import lax
from jax.experimental import pallas as pl
from jax.experimental.pallas import tpu as pltpu
from jax.experimental.pallas import tpu_sc as plsc

N = 10000
E = 160000
NC = 2
NS = 16
L = 16

N_PAD = 10240
E_PAD = NC * NS * 5120
EPT = E_PAD // (NC * NS)
CH = 128
NCHUNK = EPT // CH
ROWS_PER_TILE = N_PAD // NS
ZROWS = 64


def _mesh():
  return plsc.VectorSubcoreMesh(
      core_axis_name="c", subcore_axis_name="s", num_cores=NC,
      num_subcores=NS)


_SC_PARAMS = pltpu.CompilerParams(
    needs_layout_passes=False, use_tc_tiling_on_sc=False)


def _tile_edge_base(c, s):
  return (s * NC + c) * EPT


def _deg_body(src_hbm, wm16_hbm, out_hbm, src_v, val_v, zbuf, part):
  c = lax.axis_index("c")
  s = lax.axis_index("s")

  def zero_z(r, _):
    for q in range(1):
      zbuf[r, pl.ds(0, 16)] = jnp.zeros((16,), jnp.float32)
    return 0
  lax.fori_loop(0, ZROWS, zero_z, 0)
  for z in range(ROWS_PER_TILE // ZROWS):
    pltpu.sync_copy(zbuf, part.at[pl.ds(s * ROWS_PER_TILE + z * ZROWS, ZROWS)])
  plsc.subcore_barrier()

  ebase = _tile_edge_base(c, s)

  def chunk(j, _):
    off = ebase + j * CH
    pltpu.sync_copy(src_hbm.at[pl.ds(off, CH)], src_v)
    pltpu.sync_copy(wm16_hbm.at[pl.ds(off, CH)], val_v)
    pltpu.sync_copy(val_v, part.at[src_v], add=True)
    return 0
  lax.fori_loop(0, NCHUNK, chunk, 0)
  plsc.subcore_barrier()

  for z in range(ROWS_PER_TILE // ZROWS):
    r0 = s * ROWS_PER_TILE + z * ZROWS
    pltpu.sync_copy(part.at[pl.ds(r0, ZROWS)], out_hbm.at[c].at[pl.ds(r0, ZROWS)])


_deg_call = pl.kernel(
    _deg_body,
    out_type=jax.ShapeDtypeStruct((NC, N_PAD, 16), jnp.float32),
    mesh=_mesh(),
    compiler_params=_SC_PARAMS,
    scratch_types=[
        pltpu.VMEM((CH,), jnp.int32),
        pltpu.VMEM((CH, 16), jnp.float32),
        pltpu.VMEM((ZROWS, 16), jnp.float32),
        pltpu.VMEM_SHARED((N_PAD, 16), jnp.float32),
    ],
)


def _norm_body(src_hbm, dst_hbm, wm_hbm, dinv_hbm, nrm1_hbm, nrm2_hbm,
               src_v, dst_v, wm_v, n1_v, n2_v, dinv_v):
  c = lax.axis_index("c")
  s = lax.axis_index("s")
  pltpu.sync_copy(dinv_hbm, dinv_v)
  ebase = _tile_edge_base(c, s)

  def chunk(j, _):
    off = ebase + j * CH
    pltpu.sync_copy(src_hbm.at[pl.ds(off, CH)], src_v)
    pltpu.sync_copy(dst_hbm.at[pl.ds(off, CH)], dst_v)
    pltpu.sync_copy(wm_hbm.at[pl.ds(off, CH)], wm_v)

    def grp(g, _):
      s16 = src_v[pl.ds(g * 16, 16)]
      d16 = dst_v[pl.ds(g * 16, 16)]
      w16 = wm_v[pl.ds(g * 16, 16)]
      ds_ = plsc.load_gather(dinv_v, [s16])
      dd_ = plsc.load_gather(dinv_v, [d16])
      n1 = -(ds_ * w16 * dd_)
      n1_v[pl.ds(g * 16, 16)] = n1
      n2_v[pl.ds(g * 16, 16)] = n1 + n1
      return 0
    lax.fori_loop(0, CH // 16, grp, 0)
    pltpu.sync_copy(n1_v, nrm1_hbm.at[pl.ds(off, CH)])
    pltpu.sync_copy(n2_v, nrm2_hbm.at[pl.ds(off, CH)])
    return 0
  lax.fori_loop(0, NCHUNK, chunk, 0)


_norm_call = pl.kernel(
    _norm_body,
    out_type=(jax.ShapeDtypeStruct((E_PAD,), jnp.float32),
              jax.ShapeDtypeStruct((E_PAD,), jnp.float32)),
    mesh=_mesh(),
    compiler_params=_SC_PARAMS,
    scratch_types=[
        pltpu.VMEM((CH,), jnp.int32),
        pltpu.VMEM((CH,), jnp.int32),
        pltpu.VMEM((CH,), jnp.float32),
        pltpu.VMEM((CH,), jnp.float32),
        pltpu.VMEM((CH,), jnp.float32),
        pltpu.VMEM((N_PAD,), jnp.float32),
    ],
)


def _make_prop(F):
  def body(h_hbm, src_hbm, dst_hbm, nrm_hbm, out_hbm,
           sb0, sb1, sb2, db0, db1, db2, nrmb, G, zbuf, part,
           ig0, ig1, ig2, gs0, gs1, ss0, ss1, wsem):
    c = lax.axis_index("c")
    s = lax.axis_index("s")
    srcb = (sb0, sb1, sb2)
    dstb = (db0, db1, db2)
    isems = (ig0, ig1, ig2)
    gsems = (gs0, gs1)
    ssems = (ss0, ss1)

    def zero_z(r, _):
      for q in range(F // 16):
        zbuf[r, pl.ds(q * 16, 16)] = jnp.zeros((16,), jnp.float32)
      return 0
    lax.fori_loop(0, ZROWS, zero_z, 0)
    for z in range(ROWS_PER_TILE // ZROWS):
      pltpu.async_copy(zbuf,
                       part.at[pl.ds(s * ROWS_PER_TILE + z * ZROWS, ZROWS)],
                       wsem)
    for z in range(ROWS_PER_TILE // ZROWS):
      pltpu.make_async_copy(zbuf,
                            part.at[pl.ds(s * ROWS_PER_TILE + z * ZROWS,
                                          ZROWS)], wsem).wait()
    plsc.subcore_barrier()

    ebase = _tile_edge_base(c, s)

    def start_idx(j, t):
      off = ebase + j * CH
      pltpu.async_copy(src_hbm.at[pl.ds(off, CH)], srcb[t], isems[t])
      pltpu.async_copy(dst_hbm.at[pl.ds(off, CH)], dstb[t], isems[t])
      pltpu.async_copy(nrm_hbm.at[pl.ds(off, CH)],
                       nrmb.at[pl.ds(t * CH, CH)], isems[t])

    def wait_idx(t):
      pltpu.make_async_copy(src_hbm.at[pl.ds(0, CH)], srcb[t],
                            isems[t]).wait()
      pltpu.make_async_copy(dst_hbm.at[pl.ds(0, CH)], dstb[t],
                            isems[t]).wait()
      pltpu.make_async_copy(nrm_hbm.at[pl.ds(0, CH)],
                            nrmb.at[pl.ds(t * CH, CH)], isems[t]).wait()

    def start_gather(t, b):
      pltpu.async_copy(h_hbm.at[srcb[t]], G.at[b], gsems[b])

    def wait_gather(t, b):
      pltpu.make_async_copy(h_hbm.at[srcb[0]], G.at[b], gsems[b]).wait()

    start_idx(0, 0)
    start_idx(1, 1)
    wait_idx(0)
    start_gather(0, 0)

    def chunk(j, _):
      b = lax.rem(j, 2)
      t = lax.rem(j, 3)
      t1 = lax.rem(j + 1, 3)
      t2 = lax.rem(j + 2, 3)

      for bb in range(2):
        @pl.when(b == bb)
        def _():
          wait_gather(t, bb)

          @pl.when(j >= 1)
          def _():
            pltpu.make_async_copy(G.at[1 - bb], part.at[dstb[0]],
                                  ssems[1 - bb]).wait()

          @pl.when(j + 2 <= NCHUNK - 1)
          def _():
            for tt in range(3):
              @pl.when(t2 == tt)
              def _():
                start_idx(j + 2, tt)

          @pl.when(j + 1 <= NCHUNK - 1)
          def _():
            for tt in range(3):
              @pl.when(t1 == tt)
              def _():
                wait_idx(tt)
                start_gather(tt, 1 - bb)

          nbase = t * CH

          @plsc.parallel_loop(0, CH, 1, unroll=4)
          def _(e):
            esplat = jnp.full((16,), 0, jnp.int32) + nbase + e
            row = plsc.load_gather(nrmb, [esplat])
            for q in range(F // 16):
              G[bb, e, pl.ds(q * 16, 16)] = (
                  G[bb, e, pl.ds(q * 16, 16)] * row)

          for tt in range(3):
            @pl.when(t == tt)
            def _():
              pltpu.async_copy(G.at[bb], part.at[dstb[tt]], ssems[bb],
                               add=True)
      return 0
    lax.fori_loop(0, NCHUNK, chunk, 0)

    lastb = (NCHUNK - 1) % 2
    pltpu.make_async_copy(G.at[lastb], part.at[dstb[0]],
                          ssems[lastb]).wait()
    plsc.subcore_barrier()

    for z in range(ROWS_PER_TILE // ZROWS):
      r0 = s * ROWS_PER_TILE + z * ZROWS
      pltpu.async_copy(part.at[pl.ds(r0, ZROWS)],
                       out_hbm.at[c].at[pl.ds(r0, ZROWS)], wsem)
    for z in range(ROWS_PER_TILE // ZROWS):
      r0 = s * ROWS_PER_TILE + z * ZROWS
      pltpu.make_async_copy(part.at[pl.ds(r0, ZROWS)],
                            out_hbm.at[c].at[pl.ds(r0, ZROWS)], wsem).wait()

  return pl.kernel(
      body,
      out_type=jax.ShapeDtypeStruct((NC, N_PAD, F), jnp.float32),
      mesh=_mesh(),
      compiler_params=_SC_PARAMS,
      scratch_types=[
          pltpu.VMEM((CH,), jnp.int32),
          pltpu.VMEM((CH,), jnp.int32),
          pltpu.VMEM((CH,), jnp.int32),
          pltpu.VMEM((CH,), jnp.int32),
          pltpu.VMEM((CH,), jnp.int32),
          pltpu.VMEM((CH,), jnp.int32),
          pltpu.VMEM((3 * CH,), jnp.float32),
          pltpu.VMEM((2, CH, F), jnp.float32),
          pltpu.VMEM((ZROWS, F), jnp.float32),
          pltpu.VMEM_SHARED((N_PAD, F), jnp.float32),
          pltpu.SemaphoreType.DMA,
          pltpu.SemaphoreType.DMA,
          pltpu.SemaphoreType.DMA,
          pltpu.SemaphoreType.DMA,
          pltpu.SemaphoreType.DMA,
          pltpu.SemaphoreType.DMA,
          pltpu.SemaphoreType.DMA,
          pltpu.SemaphoreType.DMA,
      ],
  )


_prop128 = _make_prop(128)
_prop64 = _make_prop(64)
_prop_by_f = {128: _prop128, 64: _prop64}


def _dinv_body(dp_ref, o_ref):
  d = dp_ref[0, :, 0] + dp_ref[1, :, 0]
  o_ref[...] = jnp.where(
      d > 0.0, lax.rsqrt(jnp.maximum(d, 1e-12)), 0.0)


def _dinv_call(deg_parts):
  return pl.pallas_call(
      _dinv_body,
      out_shape=jax.ShapeDtypeStruct((N_PAD,), jnp.float32),
  )(deg_parts)


def _combine_body(coef_ref, p_ref, t0_ref, w_ref, acc_ref, t2_ref, acc2_ref):
  a = coef_ref[0, 0]
  b = coef_ref[0, 1]
  t2 = a * (p_ref[0] + p_ref[1]) - b * t0_ref[...]
  t2_ref[...] = t2
  acc2_ref[...] = acc_ref[...] + jnp.dot(
      t2, w_ref[...], preferred_element_type=jnp.float32)


def _combine(coef, parts, t0, w, acc):
  F = t0.shape[1]
  FO = w.shape[1]
  R = 1024
  grid = (N_PAD // R,)
  return pl.pallas_call(
      _combine_body,
      grid=grid,
      in_specs=[
          pl.BlockSpec(memory_space=pltpu.SMEM),
          pl.BlockSpec((NC, R, F), lambda i: (0, i, 0)),
          pl.BlockSpec((R, F), lambda i: (i, 0)),
          pl.BlockSpec((F, FO), lambda i: (0, 0)),
          pl.BlockSpec((R, FO), lambda i: (i, 0)),
      ],
      out_specs=[
          pl.BlockSpec((R, F), lambda i: (i, 0)),
          pl.BlockSpec((R, FO), lambda i: (i, 0)),
      ],
      out_shape=[
          jax.ShapeDtypeStruct((N_PAD, F), jnp.float32),
          jax.ShapeDtypeStruct((N_PAD, FO), jnp.float32),
      ],
  )(coef, parts, t0, w, acc)


def _silu_body(acc_ref, b_ref, o_ref):
  x = acc_ref[...] + b_ref[...]
  o_ref[...] = x * (1.0 / (1.0 + jnp.exp(-x)))


def _silu(acc, b2d):
  return pl.pallas_call(
      _silu_body,
      out_shape=jax.ShapeDtypeStruct(acc.shape, jnp.float32),
  )(acc, b2d)


def _sigmoid_body(acc_ref, o_ref):
  x = acc_ref[...]
  o_ref[...] = 1.0 / (1.0 + jnp.exp(-x))


def _sigmoid(acc):
  return pl.pallas_call(
      _sigmoid_body,
      out_shape=jax.ShapeDtypeStruct(acc.shape, jnp.float32),
  )(acc)


def _cheb_layer(h, w_stack, bias, src_p, dst_p, nrm1, nrm2):
  K, F, FO = w_stack.shape
  prop = _prop_by_f.get(F)
  zparts = jnp.zeros((NC, N_PAD, F), jnp.float32)
  zacc = jnp.zeros((N_PAD, FO), jnp.float32)
  c_id = jnp.array([[0.0, -1.0]], jnp.float32)
  c_t1 = jnp.array([[1.0, 0.0]], jnp.float32)
  c_rec = jnp.array([[2.0, 1.0]], jnp.float32)

  _, acc = _combine(c_id, zparts, h, w_stack[0], zacc)
  if K > 1:
    parts = prop(h, src_p, dst_p, nrm1)
    t1, acc = _combine(c_t1, parts, h, w_stack[1], acc)
    if K > 2:
      def body(carry, wk):
        t0, t1, acc = carry
        parts = prop(t1, src_p, dst_p, nrm2)
        t2, acc = _combine(c_rec, parts, t0, wk, acc)
        return (t1, t2, acc), None
      (_, _, acc), _ = lax.scan(body, (h, t1, acc), w_stack[2:])
  if bias is not None:
    return _silu(acc, bias)
  return acc


def _pad_w(w, fi, fo):
  K = w.shape[0]
  out = jnp.zeros((K, fi, fo), jnp.float32)
  return out.at[:, : w.shape[1], : w.shape[2]].set(w)


def kernel(x, edge_index, weight, W1, b1, W2, b2, W3, b3, W4):
  src = edge_index[0]
  dst = edge_index[1]
  wm = jnp.where(src != dst, weight, 0.0)

  pad = E_PAD - E
  src_p = jnp.concatenate([src, jnp.zeros((pad,), jnp.int32)])
  dst_p = jnp.concatenate([dst, jnp.zeros((pad,), jnp.int32)])
  wm_p = jnp.concatenate([wm, jnp.zeros((pad,), jnp.float32)])
  wm16 = jnp.broadcast_to(wm_p[:, None], (E_PAD, 16))

  deg_parts = _deg_call(src_p, wm16)
  dinv = _dinv_call(deg_parts)
  nrm1, nrm2 = _norm_call(src_p, dst_p, wm_p, dinv)

  h = jnp.zeros((N_PAD, 128), jnp.float32).at[:N].set(x)

  w1 = _pad_w(W1, 128, 64)
  w2 = _pad_w(W2, 64, 64)
  w3 = _pad_w(W3, 64, 32)
  w4 = _pad_w(W4, 32, 128)
  b1p = jnp.zeros((1, 64), jnp.float32).at[0, :60].set(b1)
  b2p = jnp.zeros((1, 64), jnp.float32).at[0, :60].set(b2)
  b3p = jnp.zeros((1, 32), jnp.float32).at[0, :30].set(b3)

  h = _cheb_layer(h, w1, b1p, src_p, dst_p, nrm1, nrm2)
  h = _cheb_layer(h, w2, b2p, src_p, dst_p, nrm1, nrm2)
  h = _cheb_layer(h, w3, b3p, src_p, dst_p, nrm1, nrm2)
  h = _cheb_layer(h, w4, None, src_p, dst_p, nrm1, nrm2)
  out = _sigmoid(h)
  return out[:N, 0:1]
